# baseline (speedup 1.0000x reference)
import functools

import jax
import jax.numpy as jnp
from jax import lax
from jax.experimental import pallas as pl
from jax.experimental.pallas import tpu as pltpu
from jax.experimental.pallas import tpu_sc as plsc

N = 10000
E = 320000
D = 128
NC = 2
NS = 16
L = 16
NW = NC * NS
CH = 128
N_CHUNKS = E // CH
N_PAD = 10240
RPT = N_PAD // NS

_mesh = plsc.VectorSubcoreMesh(core_axis_name="c", subcore_axis_name="s",
                               num_cores=NC, num_subcores=NS)


def _worker_chunk_range(wid):
    c0 = (wid * N_CHUNKS) // NW
    c1 = ((wid + 1) * N_CHUNKS) // NW
    return c0, c1


def _fill_rows(ref, value):
    vec = jnp.full((L,), value, dtype=jnp.float32)
    width = ref.shape[-1]

    def body(r, _):
        for j in range(width // L):
            ref[r, pl.ds(j * L, L)] = vec
        return _

    lax.fori_loop(0, ref.shape[0], body, None)


CPW = 80
HPW = CPW // 2
NB = 2
ND = 4


@functools.partial(
    pl.kernel,
    out_type=jax.ShapeDtypeStruct((NC, N_PAD, D), jnp.float32),
    mesh=_mesh,
    scratch_types=[
        pltpu.VMEM((CPW, CH), jnp.int32),
        pltpu.VMEM((CH, D), jnp.float32),
        pltpu.VMEM_SHARED((N_PAD, D), jnp.float32),
        pltpu.SemaphoreType.DMA((ND,)),
    ],
)
def _deg_kernel(idxm_hbm, out_hbm, idx_all, cbuf, tab, ssem):
    cid = lax.axis_index("c")
    tid = lax.axis_index("s")
    wid = tid * NC + cid
    base = tid * RPT

    _fill_rows(cbuf, 0.0)
    for j in range(RPT // CH):
        pltpu.sync_copy(cbuf, tab.at[pl.ds(base + j * CH, CH)])
    _fill_rows(cbuf, 1.0)
    plsc.subcore_barrier()

    pltpu.sync_copy(idxm_hbm.at[wid], idx_all)

    def body(j, _):
        b = lax.bitwise_and(j, ND - 1)

        @pl.when(j >= ND)
        def _drain():
            pltpu.make_async_copy(cbuf, tab.at[idx_all.at[j - ND]],
                                  ssem.at[b]).wait()

        pltpu.async_copy(cbuf, tab.at[idx_all.at[j]], ssem.at[b], add=True)
        return _

    lax.fori_loop(0, CPW, body, None)
    for d in range(ND):
        pltpu.make_async_copy(cbuf, tab.at[idx_all.at[CPW - ND + d]],
                              ssem.at[d]).wait()
    plsc.subcore_barrier()

    pltpu.sync_copy(tab.at[pl.ds(base, RPT)],
                    out_hbm.at[cid, pl.ds(base, RPT)])


@functools.partial(
    pl.kernel,
    out_type=jax.ShapeDtypeStruct((NC, N_PAD, D), jnp.float32),
    mesh=_mesh,
    scratch_types=[
        pltpu.VMEM((HPW, CH), jnp.int32),
        pltpu.VMEM((HPW, CH), jnp.int32),
        pltpu.VMEM((NB, CH, D), jnp.float32),
        pltpu.VMEM_SHARED((N_PAD, D), jnp.float32),
        pltpu.SemaphoreType.DMA((NB,)),
    ],
)
def _agg_kernel(y_hbm, srcm_hbm, dstm_hbm, out_hbm, sidx, didx, rows, acc, gsem):
    cid = lax.axis_index("c")
    tid = lax.axis_index("s")
    wid = tid * NC + cid
    base = tid * RPT

    _fill_rows(rows.at[0], 0.0)
    for j in range(RPT // CH):
        pltpu.sync_copy(rows.at[0], acc.at[pl.ds(base + j * CH, CH)])
    plsc.subcore_barrier()

    for h in range(2):
        pltpu.sync_copy(srcm_hbm.at[wid, pl.ds(h * HPW, HPW)], sidx)
        pltpu.sync_copy(dstm_hbm.at[wid, pl.ds(h * HPW, HPW)], didx)
        for b in range(NB):
            pltpu.async_copy(y_hbm.at[sidx.at[b]], rows.at[b], gsem.at[b])

        def body(j, _):
            b = lax.bitwise_and(j, NB - 1)
            pltpu.make_async_copy(y_hbm.at[sidx.at[j]], rows.at[b],
                                  gsem.at[b]).wait()
            pltpu.sync_copy(rows.at[b], acc.at[didx.at[j]], add=True)
            nxt = j + NB

            @pl.when(nxt < HPW)
            def _issue():
                pltpu.async_copy(y_hbm.at[sidx.at[nxt]], rows.at[b], gsem.at[b])

            return _

        lax.fori_loop(0, HPW, body, None)
    plsc.subcore_barrier()

    pltpu.sync_copy(acc.at[pl.ds(base, RPT)],
                    out_hbm.at[cid, pl.ds(base, RPT)])


def _norm_from_deg(deg_cols):
    deg = deg_cols[:, 0:1] + deg_cols[:, 1:2]
    return lax.rsqrt(jnp.where(deg > 0, deg, 1.0))


_MB = 2000


def _prescale_body(x_ref, dsrc_ref, o_ref):
    o_ref[...] = x_ref[...] * _norm_from_deg(dsrc_ref[...])


def _prescale(x, dsrc_t):
    grid = N // _MB
    return pl.pallas_call(
        _prescale_body,
        grid=(grid,),
        in_specs=[
            pl.BlockSpec((_MB, D), lambda i: (i, 0)),
            pl.BlockSpec((_MB, 2), lambda i: (i, 0)),
        ],
        out_specs=pl.BlockSpec((_MB, D), lambda i: (i, 0)),
        out_shape=jax.ShapeDtypeStruct((N, D), jnp.float32),
    )(x, dsrc_t)


def _make_mm_body(scale_out):
    def body(p_ref, ddst_ref, dsrc_ref, w_ref, b_ref, o_ref):
        agg = (p_ref[0] + p_ref[1]) * _norm_from_deg(ddst_ref[...])
        h = jnp.dot(agg, w_ref[...], preferred_element_type=jnp.float32)
        h = h + b_ref[...]
        if scale_out:
            h = h * _norm_from_deg(dsrc_ref[...])
        o_ref[...] = h
    return body


def _mm(p, ddst_t, dsrc_t, w, b, scale_out):
    grid = N // _MB
    return pl.pallas_call(
        _make_mm_body(scale_out),
        grid=(grid,),
        in_specs=[
            pl.BlockSpec((NC, _MB, D), lambda i: (0, i, 0)),
            pl.BlockSpec((_MB, 2), lambda i: (i, 0)),
            pl.BlockSpec((_MB, 2), lambda i: (i, 0)),
            pl.BlockSpec((D, D), lambda i: (0, 0)),
            pl.BlockSpec((1, D), lambda i: (0, 0)),
        ],
        out_specs=pl.BlockSpec((_MB, D), lambda i: (i, 0)),
        out_shape=jax.ShapeDtypeStruct((N, D), jnp.float32),
    )(p, ddst_t, dsrc_t, w, b)


def kernel(in_feat, edge_index, W1, b1, W2, b2):
    src = edge_index[0].astype(jnp.int32)
    dst = edge_index[1].astype(jnp.int32)
    pad = NW * CPW * CH - E
    trash = N + jnp.arange(pad, dtype=jnp.int32) % (N_PAD - N)
    srcm = jnp.pad(src, (0, pad)).reshape(NW, CPW, CH)
    dstm = jnp.concatenate([dst, trash]).reshape(NW, CPW, CH)
    srcd = jnp.concatenate([src, trash]).reshape(NW, CPW, CH)

    dsrc_t = _deg_kernel(srcd)[:, :N, 0].T
    ddst_t = _deg_kernel(dstm)[:, :N, 0].T

    y0 = _prescale(in_feat, dsrc_t)
    p1 = _agg_kernel(y0, srcm, dstm)
    y1 = _mm(p1, ddst_t, dsrc_t, W1, b1.reshape(1, D), scale_out=True)
    p2 = _agg_kernel(y1, srcm, dstm)
    h2 = _mm(p2, ddst_t, dsrc_t, W2, b2.reshape(1, D), scale_out=False)
    return h2


# speedup vs baseline: 1.0536x; 1.0536x over previous
import functools

import jax
import jax.numpy as jnp
from jax import lax
from jax.experimental import pallas as pl
from jax.experimental.pallas import tpu as pltpu
from jax.experimental.pallas import tpu_sc as plsc

N = 10000
E = 320000
D = 128
NC = 2
NS = 16
L = 16
NW = NC * NS
CH = 128
N_CHUNKS = E // CH
N_PAD = 10240
RPT = N_PAD // NS

_mesh = plsc.VectorSubcoreMesh(core_axis_name="c", subcore_axis_name="s",
                               num_cores=NC, num_subcores=NS)


def _worker_chunk_range(wid):
    c0 = (wid * N_CHUNKS) // NW
    c1 = ((wid + 1) * N_CHUNKS) // NW
    return c0, c1


def _fill_rows(ref, value):
    vec = jnp.full((L,), value, dtype=jnp.float32)
    width = ref.shape[-1]

    def body(r, _):
        for j in range(width // L):
            ref[r, pl.ds(j * L, L)] = vec
        return _

    lax.fori_loop(0, ref.shape[0], body, None)


CPW = 80
SEG = 32
C0 = 128
C1 = 32
NB = 2
ND = 4


@functools.partial(
    pl.kernel,
    out_type=jax.ShapeDtypeStruct((NC, N_PAD, D), jnp.float32),
    mesh=_mesh,
    scratch_types=[
        pltpu.VMEM((CPW, CH), jnp.int32),
        pltpu.VMEM((CH, D), jnp.float32),
        pltpu.VMEM_SHARED((N_PAD, D), jnp.float32),
        pltpu.SemaphoreType.DMA((ND,)),
    ],
)
def _deg_kernel(idxm_hbm, out_hbm, idx_all, cbuf, tab, ssem):
    cid = lax.axis_index("c")
    tid = lax.axis_index("s")
    wid = tid * NC + cid
    base = tid * RPT

    _fill_rows(cbuf, 0.0)
    for j in range(RPT // CH):
        pltpu.sync_copy(cbuf, tab.at[pl.ds(base + j * CH, CH)])
    _fill_rows(cbuf, 1.0)
    plsc.subcore_barrier()

    pltpu.sync_copy(idxm_hbm.at[wid], idx_all)

    def body(j, _):
        b = lax.bitwise_and(j, ND - 1)

        @pl.when(j >= ND)
        def _drain():
            pltpu.make_async_copy(cbuf, tab.at[idx_all.at[j - ND]],
                                  ssem.at[b]).wait()

        pltpu.async_copy(cbuf, tab.at[idx_all.at[j]], ssem.at[b], add=True)
        return _

    lax.fori_loop(0, CPW, body, None)
    for d in range(ND):
        pltpu.make_async_copy(cbuf, tab.at[idx_all.at[CPW - ND + d]],
                              ssem.at[d]).wait()
    plsc.subcore_barrier()

    pltpu.sync_copy(tab.at[pl.ds(base, RPT)],
                    out_hbm.at[cid, pl.ds(base, RPT)])


@functools.partial(
    pl.kernel,
    out_type=jax.ShapeDtypeStruct((NC, N_PAD, D), jnp.float32),
    mesh=_mesh,
    scratch_types=[
        pltpu.VMEM((SEG, CH), jnp.int32),
        pltpu.VMEM((SEG, CH), jnp.int32),
        pltpu.VMEM((NB, CH, D), jnp.float32),
        pltpu.VMEM_SHARED((N_PAD, D), jnp.float32),
        pltpu.SemaphoreType.DMA((NB,)),
    ],
)
def _agg_kernel(y_hbm, s0_hbm, d0_hbm, s1_hbm, d1_hbm, out_hbm,
                sidx, didx, rows, acc, gsem):
    cid = lax.axis_index("c")
    tid = lax.axis_index("s")
    base = tid * RPT

    _fill_rows(rows.at[0], 0.0)
    for j in range(RPT // CH):
        pltpu.sync_copy(rows.at[0], acc.at[pl.ds(base + j * CH, CH)])
    plsc.subcore_barrier()

    def run(sm, dm, n_seg):
        for g in range(n_seg):
            pltpu.sync_copy(sm.at[tid, pl.ds(g * SEG, SEG)], sidx)
            pltpu.sync_copy(dm.at[tid, pl.ds(g * SEG, SEG)], didx)
            for b in range(NB):
                pltpu.async_copy(y_hbm.at[sidx.at[b]], rows.at[b], gsem.at[b])

            def body(j, _):
                b = lax.bitwise_and(j, NB - 1)
                pltpu.make_async_copy(y_hbm.at[sidx.at[j]], rows.at[b],
                                      gsem.at[b]).wait()
                pltpu.sync_copy(rows.at[b], acc.at[didx.at[j]], add=True)
                nxt = j + NB

                @pl.when(nxt < SEG)
                def _issue():
                    pltpu.async_copy(y_hbm.at[sidx.at[nxt]], rows.at[b],
                                     gsem.at[b])

                return _

            lax.fori_loop(0, SEG, body, None)

    @pl.when(cid == 0)
    def _c0():
        run(s0_hbm, d0_hbm, C0 // SEG)

    @pl.when(cid == 1)
    def _c1():
        run(s1_hbm, d1_hbm, C1 // SEG)

    plsc.subcore_barrier()

    pltpu.sync_copy(acc.at[pl.ds(base, RPT)],
                    out_hbm.at[cid, pl.ds(base, RPT)])


def _norm_from_deg(deg_cols):
    deg = deg_cols[:, 0:1] + deg_cols[:, 1:2]
    return lax.rsqrt(jnp.where(deg > 0, deg, 1.0))


_MB = 2000


def _prescale_body(x_ref, dsrc_ref, o_ref):
    o_ref[...] = x_ref[...] * _norm_from_deg(dsrc_ref[...])


def _prescale(x, dsrc_t):
    grid = N // _MB
    return pl.pallas_call(
        _prescale_body,
        grid=(grid,),
        in_specs=[
            pl.BlockSpec((_MB, D), lambda i: (i, 0)),
            pl.BlockSpec((_MB, 2), lambda i: (i, 0)),
        ],
        out_specs=pl.BlockSpec((_MB, D), lambda i: (i, 0)),
        out_shape=jax.ShapeDtypeStruct((N, D), jnp.float32),
    )(x, dsrc_t)


def _make_mm_body(scale_out):
    def body(p_ref, ddst_ref, dsrc_ref, w_ref, b_ref, o_ref):
        agg = (p_ref[0] + p_ref[1]) * _norm_from_deg(ddst_ref[...])
        h = jnp.dot(agg, w_ref[...], preferred_element_type=jnp.float32)
        h = h + b_ref[...]
        if scale_out:
            h = h * _norm_from_deg(dsrc_ref[...])
        o_ref[...] = h
    return body


def _mm(p, ddst_t, dsrc_t, w, b, scale_out):
    grid = N // _MB
    return pl.pallas_call(
        _make_mm_body(scale_out),
        grid=(grid,),
        in_specs=[
            pl.BlockSpec((NC, _MB, D), lambda i: (0, i, 0)),
            pl.BlockSpec((_MB, 2), lambda i: (i, 0)),
            pl.BlockSpec((_MB, 2), lambda i: (i, 0)),
            pl.BlockSpec((D, D), lambda i: (0, 0)),
            pl.BlockSpec((1, D), lambda i: (0, 0)),
        ],
        out_specs=pl.BlockSpec((_MB, D), lambda i: (i, 0)),
        out_shape=jax.ShapeDtypeStruct((N, D), jnp.float32),
    )(p, ddst_t, dsrc_t, w, b)


def kernel(in_feat, edge_index, W1, b1, W2, b2):
    src = edge_index[0].astype(jnp.int32)
    dst = edge_index[1].astype(jnp.int32)
    pad = NW * CPW * CH - E
    trash = N + jnp.arange(pad, dtype=jnp.int32) % (N_PAD - N)
    srcf = jnp.pad(src, (0, pad))
    dstf = jnp.concatenate([dst, trash])
    ne0 = NS * C0 * CH
    s0 = srcf[:ne0].reshape(NS, C0, CH)
    d0 = dstf[:ne0].reshape(NS, C0, CH)
    s1 = srcf[ne0:].reshape(NS, C1, CH)
    d1 = dstf[ne0:].reshape(NS, C1, CH)
    srcd = jnp.concatenate([src, trash]).reshape(NW, CPW, CH)
    dstm = dstf.reshape(NW, CPW, CH)

    dsrc_t = _deg_kernel(srcd)[:, :N, 0].T
    ddst_t = _deg_kernel(dstm)[:, :N, 0].T

    y0 = _prescale(in_feat, dsrc_t)
    p1 = _agg_kernel(y0, s0, d0, s1, d1)
    y1 = _mm(p1, ddst_t, dsrc_t, W1, b1.reshape(1, D), scale_out=True)
    p2 = _agg_kernel(y1, s0, d0, s1, d1)
    h2 = _mm(p2, ddst_t, dsrc_t, W2, b2.reshape(1, D), scale_out=False)
    return h2


# speedup vs baseline: 2.2142x; 2.1016x over previous
import functools

import jax
import jax.numpy as jnp
from jax import lax
from jax.experimental import pallas as pl
from jax.experimental.pallas import tpu as pltpu
from jax.experimental.pallas import tpu_sc as plsc

N = 10000
E = 320000
D = 128
NC = 2
NS = 16
L = 16
NW = NC * NS
CH = 128
N_CHUNKS = E // CH
N_PAD = 10240
RPT = N_PAD // NS

_mesh = plsc.VectorSubcoreMesh(core_axis_name="c", subcore_axis_name="s",
                               num_cores=NC, num_subcores=NS)


def _worker_chunk_range(wid):
    c0 = (wid * N_CHUNKS) // NW
    c1 = ((wid + 1) * N_CHUNKS) // NW
    return c0, c1


def _fill_rows(ref, value):
    vec = jnp.full((L,), value, dtype=jnp.float32)
    width = ref.shape[-1]

    def body(r, _):
        for j in range(width // L):
            ref[r, pl.ds(j * L, L)] = vec
        return _

    lax.fori_loop(0, ref.shape[0], body, None)


CPW = 80
SEG = 32
C0 = 128
C1 = 32
NB = 2
ND = 4


@functools.partial(
    pl.kernel,
    out_type=jax.ShapeDtypeStruct((NC, N_PAD, D), jnp.float32),
    mesh=_mesh,
    scratch_types=[
        pltpu.VMEM((CPW, CH), jnp.int32),
        pltpu.VMEM((CH, D), jnp.float32),
        pltpu.VMEM_SHARED((N_PAD, D), jnp.float32),
        pltpu.SemaphoreType.DMA((ND,)),
    ],
)
def _deg_kernel(idxm_hbm, out_hbm, idx_all, cbuf, tab, ssem):
    cid = lax.axis_index("c")
    tid = lax.axis_index("s")
    wid = tid * NC + cid
    base = tid * RPT

    _fill_rows(cbuf, 0.0)
    for j in range(RPT // CH):
        pltpu.sync_copy(cbuf, tab.at[pl.ds(base + j * CH, CH)])
    _fill_rows(cbuf, 1.0)
    plsc.subcore_barrier()

    pltpu.sync_copy(idxm_hbm.at[wid], idx_all)

    def body(j, _):
        b = lax.bitwise_and(j, ND - 1)

        @pl.when(j >= ND)
        def _drain():
            pltpu.make_async_copy(cbuf, tab.at[idx_all.at[j - ND]],
                                  ssem.at[b]).wait()

        pltpu.async_copy(cbuf, tab.at[idx_all.at[j]], ssem.at[b], add=True)
        return _

    lax.fori_loop(0, CPW, body, None)
    for d in range(ND):
        pltpu.make_async_copy(cbuf, tab.at[idx_all.at[CPW - ND + d]],
                              ssem.at[d]).wait()
    plsc.subcore_barrier()

    pltpu.sync_copy(tab.at[pl.ds(base, RPT)],
                    out_hbm.at[cid, pl.ds(base, RPT)])


@functools.partial(
    pl.kernel,
    out_type=jax.ShapeDtypeStruct((NC, N_PAD, D), jnp.float32),
    mesh=_mesh,
    scratch_types=[
        pltpu.VMEM((SEG, CH), jnp.int32),
        pltpu.VMEM((SEG, CH), jnp.int32),
        pltpu.VMEM((NB, CH, D), jnp.float32),
        pltpu.VMEM_SHARED((N_PAD, D), jnp.float32),
        pltpu.SemaphoreType.DMA((NB,)),
    ],
)
def _agg_kernel(y_hbm, s0_hbm, d0_hbm, s1_hbm, d1_hbm, out_hbm,
                sidx, didx, rows, acc, gsem):
    cid = lax.axis_index("c")
    tid = lax.axis_index("s")
    base = tid * RPT

    _fill_rows(rows.at[0], 0.0)
    for j in range(RPT // CH):
        pltpu.sync_copy(rows.at[0], acc.at[pl.ds(base + j * CH, CH)])
    plsc.subcore_barrier()

    def run(sm, dm, n_seg):
        for g in range(n_seg):
            pltpu.sync_copy(sm.at[tid, pl.ds(g * SEG, SEG)], sidx)
            pltpu.sync_copy(dm.at[tid, pl.ds(g * SEG, SEG)], didx)
            for b in range(NB):
                pltpu.async_copy(y_hbm.at[sidx.at[b]], rows.at[b], gsem.at[b])

            def body(j, _):
                b = lax.bitwise_and(j, NB - 1)
                pltpu.make_async_copy(y_hbm.at[sidx.at[j]], rows.at[b],
                                      gsem.at[b]).wait()
                pltpu.sync_copy(rows.at[b], acc.at[didx.at[j]], add=True)
                nxt = j + NB

                @pl.when(nxt < SEG)
                def _issue():
                    pltpu.async_copy(y_hbm.at[sidx.at[nxt]], rows.at[b],
                                     gsem.at[b])

                return _

            lax.fori_loop(0, SEG, body, None)

    @pl.when(cid == 0)
    def _c0():
        run(s0_hbm, d0_hbm, C0 // SEG)

    @pl.when(cid == 1)
    def _c1():
        run(s1_hbm, d1_hbm, C1 // SEG)

    plsc.subcore_barrier()

    pltpu.sync_copy(acc.at[pl.ds(base, RPT)],
                    out_hbm.at[cid, pl.ds(base, RPT)])


def _norm_from_deg(deg_cols):
    deg = deg_cols[:, 0:1] + deg_cols[:, 1:2]
    return lax.rsqrt(jnp.where(deg > 0, deg, 1.0))


_MB = 2000


def _prescale_body(x_ref, dsrc_ref, o_ref):
    o_ref[...] = x_ref[...] * _norm_from_deg(dsrc_ref[...])


def _prescale(x, dsrc_t):
    grid = N // _MB
    return pl.pallas_call(
        _prescale_body,
        grid=(grid,),
        in_specs=[
            pl.BlockSpec((_MB, D), lambda i: (i, 0)),
            pl.BlockSpec((_MB, 2), lambda i: (i, 0)),
        ],
        out_specs=pl.BlockSpec((_MB, D), lambda i: (i, 0)),
        out_shape=jax.ShapeDtypeStruct((N, D), jnp.float32),
    )(x, dsrc_t)


def _make_mm_body(scale_out):
    def body(p_ref, ddst_ref, dsrc_ref, w_ref, b_ref, o_ref):
        agg = (p_ref[0] + p_ref[1]) * _norm_from_deg(ddst_ref[...])
        h = jnp.dot(agg, w_ref[...], preferred_element_type=jnp.float32)
        h = h + b_ref[...]
        if scale_out:
            h = h * _norm_from_deg(dsrc_ref[...])
        o_ref[...] = h
    return body


def _mm(p, ddst_t, dsrc_t, w, b, scale_out):
    grid = N // _MB
    return pl.pallas_call(
        _make_mm_body(scale_out),
        grid=(grid,),
        in_specs=[
            pl.BlockSpec((NC, _MB, D), lambda i: (0, i, 0)),
            pl.BlockSpec((_MB, 2), lambda i: (i, 0)),
            pl.BlockSpec((_MB, 2), lambda i: (i, 0)),
            pl.BlockSpec((D, D), lambda i: (0, 0)),
            pl.BlockSpec((1, D), lambda i: (0, 0)),
        ],
        out_specs=pl.BlockSpec((_MB, D), lambda i: (i, 0)),
        out_shape=jax.ShapeDtypeStruct((N, D), jnp.float32),
    )(p, ddst_t, dsrc_t, w, b)


def kernel(in_feat, edge_index, W1, b1, W2, b2):
    src = edge_index[0].astype(jnp.int32)
    dst = edge_index[1].astype(jnp.int32)
    pad = NW * CPW * CH - E
    trash = N + jnp.arange(pad, dtype=jnp.int32) % (N_PAD - N)
    gpad = jnp.arange(pad, dtype=jnp.int32) % N
    srcf = jnp.concatenate([src, gpad])
    dstf = jnp.concatenate([dst, trash])
    ne0 = NS * C0 * CH
    s0 = srcf[:ne0].reshape(NS, C0, CH)
    d0 = dstf[:ne0].reshape(NS, C0, CH)
    s1 = srcf[ne0:].reshape(NS, C1, CH)
    d1 = dstf[ne0:].reshape(NS, C1, CH)
    srcd = jnp.concatenate([src, trash]).reshape(NW, CPW, CH)
    dstm = dstf.reshape(NW, CPW, CH)

    dsrc_t = _deg_kernel(srcd)[:, :N, 0].T
    ddst_t = _deg_kernel(dstm)[:, :N, 0].T

    y0 = _prescale(in_feat, dsrc_t)
    p1 = _agg_kernel(y0, s0, d0, s1, d1)
    y1 = _mm(p1, ddst_t, dsrc_t, W1, b1.reshape(1, D), scale_out=True)
    p2 = _agg_kernel(y1, s0, d0, s1, d1)
    h2 = _mm(p2, ddst_t, dsrc_t, W2, b2.reshape(1, D), scale_out=False)
    return h2


# speedup vs baseline: 2.5773x; 1.1640x over previous
import functools

import jax
import jax.numpy as jnp
from jax import lax
from jax.experimental import pallas as pl
from jax.experimental.pallas import tpu as pltpu
from jax.experimental.pallas import tpu_sc as plsc

N = 10000
E = 320000
D = 128
NC = 2
NS = 16
L = 16
NW = NC * NS
CH = 128
N_CHUNKS = E // CH
N_PAD = 10240
RPT = N_PAD // NS

_mesh = plsc.VectorSubcoreMesh(core_axis_name="c", subcore_axis_name="s",
                               num_cores=NC, num_subcores=NS)


def _worker_chunk_range(wid):
    c0 = (wid * N_CHUNKS) // NW
    c1 = ((wid + 1) * N_CHUNKS) // NW
    return c0, c1


def _fill_rows(ref, value):
    vec = jnp.full((L,), value, dtype=jnp.float32)
    width = ref.shape[-1]

    def body(r, _):
        for j in range(width // L):
            ref[r, pl.ds(j * L, L)] = vec
        return _

    lax.fori_loop(0, ref.shape[0], body, None)


CPW = 80
SEG = 32
C0 = 96
C1 = 64
NB = 2
ND = 4


@functools.partial(
    pl.kernel,
    out_type=jax.ShapeDtypeStruct((NC, N_PAD, D), jnp.float32),
    mesh=_mesh,
    scratch_types=[
        pltpu.VMEM((CPW, CH), jnp.int32),
        pltpu.VMEM((CH, D), jnp.float32),
        pltpu.VMEM_SHARED((N_PAD, D), jnp.float32),
        pltpu.SemaphoreType.DMA((ND,)),
    ],
)
def _deg_kernel(idxm_hbm, out_hbm, idx_all, cbuf, tab, ssem):
    cid = lax.axis_index("c")
    tid = lax.axis_index("s")
    wid = tid * NC + cid
    base = tid * RPT

    _fill_rows(cbuf, 0.0)
    for j in range(RPT // CH):
        pltpu.sync_copy(cbuf, tab.at[pl.ds(base + j * CH, CH)])
    _fill_rows(cbuf, 1.0)
    plsc.subcore_barrier()

    pltpu.sync_copy(idxm_hbm.at[wid], idx_all)

    def body(j, _):
        b = lax.bitwise_and(j, ND - 1)

        @pl.when(j >= ND)
        def _drain():
            pltpu.make_async_copy(cbuf, tab.at[idx_all.at[j - ND]],
                                  ssem.at[b]).wait()

        pltpu.async_copy(cbuf, tab.at[idx_all.at[j]], ssem.at[b], add=True)
        return _

    lax.fori_loop(0, CPW, body, None)
    for d in range(ND):
        pltpu.make_async_copy(cbuf, tab.at[idx_all.at[CPW - ND + d]],
                              ssem.at[d]).wait()
    plsc.subcore_barrier()

    pltpu.sync_copy(tab.at[pl.ds(base, RPT)],
                    out_hbm.at[cid, pl.ds(base, RPT)])


@functools.partial(
    pl.kernel,
    out_type=jax.ShapeDtypeStruct((NC, N_PAD, D), jnp.float32),
    mesh=_mesh,
    scratch_types=[
        pltpu.VMEM((SEG, CH), jnp.int32),
        pltpu.VMEM((SEG, CH), jnp.int32),
        pltpu.VMEM((NB, CH, D), jnp.float32),
        pltpu.VMEM_SHARED((N_PAD, D), jnp.float32),
        pltpu.SemaphoreType.DMA((NB,)),
    ],
)
def _agg_kernel(y_hbm, s0_hbm, d0_hbm, s1_hbm, d1_hbm, out_hbm,
                sidx, didx, rows, acc, gsem):
    cid = lax.axis_index("c")
    tid = lax.axis_index("s")
    base = tid * RPT

    _fill_rows(rows.at[0], 0.0)
    for j in range(RPT // CH):
        pltpu.sync_copy(rows.at[0], acc.at[pl.ds(base + j * CH, CH)])
    plsc.subcore_barrier()

    def run(sm, dm, n_seg):
        for g in range(n_seg):
            pltpu.sync_copy(sm.at[tid, pl.ds(g * SEG, SEG)], sidx)
            pltpu.sync_copy(dm.at[tid, pl.ds(g * SEG, SEG)], didx)
            for b in range(NB):
                pltpu.async_copy(y_hbm.at[sidx.at[b]], rows.at[b], gsem.at[b])

            def body(j, _):
                b = lax.bitwise_and(j, NB - 1)
                pltpu.make_async_copy(y_hbm.at[sidx.at[j]], rows.at[b],
                                      gsem.at[b]).wait()
                pltpu.sync_copy(rows.at[b], acc.at[didx.at[j]], add=True)
                nxt = j + NB

                @pl.when(nxt < SEG)
                def _issue():
                    pltpu.async_copy(y_hbm.at[sidx.at[nxt]], rows.at[b],
                                     gsem.at[b])

                return _

            lax.fori_loop(0, SEG, body, None)

    @pl.when(cid == 0)
    def _c0():
        run(s0_hbm, d0_hbm, C0 // SEG)

    @pl.when(cid == 1)
    def _c1():
        run(s1_hbm, d1_hbm, C1 // SEG)

    plsc.subcore_barrier()

    pltpu.sync_copy(acc.at[pl.ds(base, RPT)],
                    out_hbm.at[cid, pl.ds(base, RPT)])


def _norm_from_deg(deg_cols):
    deg = deg_cols[:, 0:1] + deg_cols[:, 1:2]
    return lax.rsqrt(jnp.where(deg > 0, deg, 1.0))


_MB = 2000


def _prescale_body(x_ref, dsrc_ref, o_ref):
    o_ref[...] = x_ref[...] * _norm_from_deg(dsrc_ref[...])


def _prescale(x, dsrc_t):
    grid = N // _MB
    return pl.pallas_call(
        _prescale_body,
        grid=(grid,),
        in_specs=[
            pl.BlockSpec((_MB, D), lambda i: (i, 0)),
            pl.BlockSpec((_MB, 2), lambda i: (i, 0)),
        ],
        out_specs=pl.BlockSpec((_MB, D), lambda i: (i, 0)),
        out_shape=jax.ShapeDtypeStruct((N, D), jnp.float32),
    )(x, dsrc_t)


def _make_mm_body(scale_out):
    def body(p_ref, ddst_ref, dsrc_ref, w_ref, b_ref, o_ref):
        agg = (p_ref[0] + p_ref[1]) * _norm_from_deg(ddst_ref[...])
        h = jnp.dot(agg, w_ref[...], preferred_element_type=jnp.float32)
        h = h + b_ref[...]
        if scale_out:
            h = h * _norm_from_deg(dsrc_ref[...])
        o_ref[...] = h
    return body


def _mm(p, ddst_t, dsrc_t, w, b, scale_out):
    grid = N // _MB
    return pl.pallas_call(
        _make_mm_body(scale_out),
        grid=(grid,),
        in_specs=[
            pl.BlockSpec((NC, _MB, D), lambda i: (0, i, 0)),
            pl.BlockSpec((_MB, 2), lambda i: (i, 0)),
            pl.BlockSpec((_MB, 2), lambda i: (i, 0)),
            pl.BlockSpec((D, D), lambda i: (0, 0)),
            pl.BlockSpec((1, D), lambda i: (0, 0)),
        ],
        out_specs=pl.BlockSpec((_MB, D), lambda i: (i, 0)),
        out_shape=jax.ShapeDtypeStruct((N, D), jnp.float32),
    )(p, ddst_t, dsrc_t, w, b)


def kernel(in_feat, edge_index, W1, b1, W2, b2):
    src = edge_index[0].astype(jnp.int32)
    dst = edge_index[1].astype(jnp.int32)
    pad = NW * CPW * CH - E
    trash = N + jnp.arange(pad, dtype=jnp.int32) % (N_PAD - N)
    gpad = jnp.arange(pad, dtype=jnp.int32) % N
    srcf = jnp.concatenate([src, gpad])
    dstf = jnp.concatenate([dst, trash])
    ne0 = NS * C0 * CH
    s0 = srcf[:ne0].reshape(NS, C0, CH)
    d0 = dstf[:ne0].reshape(NS, C0, CH)
    s1 = srcf[ne0:].reshape(NS, C1, CH)
    d1 = dstf[ne0:].reshape(NS, C1, CH)
    srcd = jnp.concatenate([src, trash]).reshape(NW, CPW, CH)
    dstm = dstf.reshape(NW, CPW, CH)

    dsrc_t = _deg_kernel(srcd)[:, :N, 0].T
    ddst_t = _deg_kernel(dstm)[:, :N, 0].T

    y0 = _prescale(in_feat, dsrc_t)
    p1 = _agg_kernel(y0, s0, d0, s1, d1)
    y1 = _mm(p1, ddst_t, dsrc_t, W1, b1.reshape(1, D), scale_out=True)
    p2 = _agg_kernel(y1, s0, d0, s1, d1)
    h2 = _mm(p2, ddst_t, dsrc_t, W2, b2.reshape(1, D), scale_out=False)
    return h2
